# baseline (speedup 1.0000x reference)
import functools

import jax
import jax.numpy as jnp
from jax import lax
from jax.experimental import pallas as pl
from jax.experimental.pallas import tpu as pltpu
from jax.experimental.pallas import tpu_sc as plsc

N_ATOMS = 50000
N_IMP = 100000
D = 128
MID = 128
K_OUT = 6

NC = 2
NS = 16
NW = NC * NS

P = 100352
R_PER_W = P // NW
CHUNK = 112
NCHUNK = R_PER_W // CHUNK


def _sc_gather_body(u_h, v_h, i0_h, i1_h, i2_h, i3_h, w_h,
                    iv, g0a, g0b, g2a, g2b, g3a, g3b, gva, gvb, wb0, wb1,
                    sg0, sg1, sw0, sw1):
    wid = lax.axis_index("s") * NC + lax.axis_index("c")
    base = wid * R_PER_W
    R = R_PER_W
    bufs = ((g0a, g2a, g3a, gva), (g0b, g2b, g3b, gvb))
    wbs = (wb0, wb1)
    sgs = (sg0, sg1)
    sws = (sw0, sw1)

    D2 = D // 2

    pltpu.sync_copy(i0_h.at[pl.ds(base, R)], iv.at[pl.ds(0, R)])
    pltpu.sync_copy(i2_h.at[pl.ds(base, R)], iv.at[pl.ds(R, R)])
    pltpu.sync_copy(i3_h.at[pl.ds(base, R)], iv.at[pl.ds(2 * R, R)])
    pltpu.sync_copy(i1_h.at[pl.ds(base, R)], iv.at[pl.ds(3 * R, R)])

    def fire(kk, b):
        g0, g2, g3, gv = bufs[b]
        pltpu.async_copy(u_h.at[iv.at[pl.ds(kk * CHUNK, CHUNK)]], g0, sgs[b])
        pltpu.async_copy(
            u_h.at[iv.at[pl.ds(R + kk * CHUNK, CHUNK)]], g2, sgs[b])
        pltpu.async_copy(
            u_h.at[iv.at[pl.ds(2 * R + kk * CHUNK, CHUNK)]], g3, sgs[b])
        pltpu.async_copy(
            v_h.at[iv.at[pl.ds(3 * R + kk * CHUNK, CHUNK)]], gv, sgs[b])

    def wait_gathers(b):
        for dst in bufs[b]:
            pltpu.make_async_copy(
                u_h.at[iv.at[pl.ds(0, CHUNK)]], dst, sgs[b]).wait()

    fire(0, 0)

    def outer(k2, carry):
        for b in (0, 1):
            kk = k2 * 2 + b

            @pl.when(kk + 1 < NCHUNK)
            def _():
                fire(kk + 1, 1 - b)

            wait_gathers(b)

            @pl.when(kk >= 2)
            def _():
                pltpu.make_async_copy(
                    wbs[b], w_h.at[pl.ds(base, CHUNK)], sws[b]).wait()

            g0, g2, g3, gv = bufs[b]
            wb = wbs[b]

            def row(r, acc):
                for j in range(D2 // 16):
                    sl = pl.ds(j * 16, 16)
                    a0 = plsc.bitcast(g0[r, sl], jnp.bfloat16)
                    a2 = plsc.bitcast(g2[r, sl], jnp.bfloat16)
                    a3 = plsc.bitcast(g3[r, sl], jnp.bfloat16)
                    av = plsc.bitcast(gv[r, sl], jnp.bfloat16)
                    wb[r, sl] = plsc.bitcast(
                        (a0 + a2) + (a3 + av), jnp.int32)
                return acc

            lax.fori_loop(0, CHUNK, row, 0, unroll=4)
            pltpu.async_copy(
                wb, w_h.at[pl.ds(base + kk * CHUNK, CHUNK)], sws[b])
        return carry

    lax.fori_loop(0, NCHUNK // 2, outer, 0)
    for b in (0, 1):
        pltpu.make_async_copy(
            wbs[b], w_h.at[pl.ds(base, CHUNK)], sws[b]).wait()


def _sc_gather_combine(u, v, i0, i1, i2, i3):
    mesh = plsc.VectorSubcoreMesh(
        core_axis_name="c", subcore_axis_name="s", num_cores=NC, num_subcores=NS
    )
    buf = pltpu.VMEM((CHUNK, D // 2), jnp.int32)
    return pl.kernel(
        _sc_gather_body,
        out_type=jax.ShapeDtypeStruct((P, D // 2), jnp.int32),
        mesh=mesh,
        compiler_params=pltpu.CompilerParams(use_tc_tiling_on_sc=False, needs_layout_passes=False),
        scratch_types=[
            pltpu.VMEM((4 * R_PER_W,), jnp.int32),
            buf, buf, buf, buf, buf, buf, buf, buf, buf, buf,
            pltpu.SemaphoreType.DMA,
            pltpu.SemaphoreType.DMA,
            pltpu.SemaphoreType.DMA,
            pltpu.SemaphoreType.DMA,
        ],
    )(u, v, i0, i1, i2, i3)


_UV_BLK = 1000


def _pack_bf16_pair(even_f32, odd_f32):
    pe = lax.bitcast_convert_type(
        even_f32.astype(jnp.bfloat16).astype(jnp.float32), jnp.uint32)
    po = lax.bitcast_convert_type(
        odd_f32.astype(jnp.bfloat16).astype(jnp.float32), jnp.uint32)
    return lax.bitcast_convert_type((pe >> 16) | po, jnp.int32)


def _uv_body(h_ref, wse_ref, wso_ref, wce_ref, wco_ref, u_ref, v_ref):
    hb = h_ref[:].astype(jnp.bfloat16)
    wse = wse_ref[:].astype(jnp.bfloat16)
    wso = wso_ref[:].astype(jnp.bfloat16)
    wce = wce_ref[:].astype(jnp.bfloat16)
    wco = wco_ref[:].astype(jnp.bfloat16)
    ue = jnp.dot(hb, wse, preferred_element_type=jnp.float32)
    uo = jnp.dot(hb, wso, preferred_element_type=jnp.float32)
    ve = jnp.dot(hb, wce, preferred_element_type=jnp.float32)
    vo = jnp.dot(hb, wco, preferred_element_type=jnp.float32)
    u_ref[:] = _pack_bf16_pair(ue, uo)
    v_ref[:] = _pack_bf16_pair(ve, vo)


def _uv_tables(h, Wse, Wso, Wce, Wco):
    D2 = D // 2
    return pl.pallas_call(
        _uv_body,
        grid=(N_ATOMS // _UV_BLK,),
        in_specs=[
            pl.BlockSpec((_UV_BLK, D), lambda i: (i, 0)),
            pl.BlockSpec((D, D2), lambda i: (0, 0)),
            pl.BlockSpec((D, D2), lambda i: (0, 0)),
            pl.BlockSpec((D, D2), lambda i: (0, 0)),
            pl.BlockSpec((D, D2), lambda i: (0, 0)),
        ],
        out_specs=[
            pl.BlockSpec((_UV_BLK, D2), lambda i: (i, 0)),
            pl.BlockSpec((_UV_BLK, D2), lambda i: (i, 0)),
        ],
        out_shape=[
            jax.ShapeDtypeStruct((N_ATOMS, D2), jnp.int32),
            jax.ShapeDtypeStruct((N_ATOMS, D2), jnp.int32),
        ],
    )(h, Wse, Wso, Wce, Wco)


_MLP_BLK = 1024


def _mlp_body(w_ref, b0e_ref, b0o_ref, w1e_ref, w1o_ref, b1_ref,
              w2_ref, b2_ref, wo_ref, bo_ref, out_ref):
    wi = w_ref[:]
    xe = lax.bitcast_convert_type(wi << 16, jnp.float32)
    xo = lax.bitcast_convert_type(
        wi & jnp.int32(-65536), jnp.float32)
    xe = jnp.maximum(xe + b0e_ref[:], 0.0).astype(jnp.bfloat16)
    xo = jnp.maximum(xo + b0o_ref[:], 0.0).astype(jnp.bfloat16)
    x = jnp.maximum(
        jnp.dot(xe, w1e_ref[:].astype(jnp.bfloat16),
                preferred_element_type=jnp.float32)
        + jnp.dot(xo, w1o_ref[:].astype(jnp.bfloat16),
                  preferred_element_type=jnp.float32)
        + b1_ref[:], 0.0).astype(jnp.bfloat16)
    x = jnp.maximum(
        jnp.dot(x, w2_ref[:].astype(jnp.bfloat16),
                preferred_element_type=jnp.float32) + b2_ref[:],
        0.0).astype(jnp.bfloat16)
    out_ref[:] = (
        jnp.dot(x, wo_ref[:].astype(jnp.bfloat16),
                preferred_element_type=jnp.float32) + bo_ref[:])


def _mlp(w, b0e, b0o, W1e, W1o, b1, W2, b2, W_out, b_out):
    kout = W_out.shape[1]
    D2 = D // 2
    return pl.pallas_call(
        _mlp_body,
        grid=(P // _MLP_BLK,),
        in_specs=[
            pl.BlockSpec((_MLP_BLK, D2), lambda i: (i, 0)),
            pl.BlockSpec((1, D2), lambda i: (0, 0)),
            pl.BlockSpec((1, D2), lambda i: (0, 0)),
            pl.BlockSpec((D2, MID), lambda i: (0, 0)),
            pl.BlockSpec((D2, MID), lambda i: (0, 0)),
            pl.BlockSpec((1, MID), lambda i: (0, 0)),
            pl.BlockSpec((MID, MID), lambda i: (0, 0)),
            pl.BlockSpec((1, MID), lambda i: (0, 0)),
            pl.BlockSpec((MID, kout), lambda i: (0, 0)),
            pl.BlockSpec((1, kout), lambda i: (0, 0)),
        ],
        out_specs=pl.BlockSpec((_MLP_BLK, kout), lambda i: (i, 0)),
        out_shape=jax.ShapeDtypeStruct((N_IMP, kout), jnp.float32),
    )(w, b0e, b0o, W1e, W1o, b1, W2, b2, W_out, b_out)


def kernel(h, idx, W0, b0, W1, b1, W2, b2, W_out, b_out):
    Ws = W0[0:D] + W0[2 * D:3 * D] + W0[3 * D:4 * D]
    Wc = W0[D:2 * D] * 3.0

    u, v = _uv_tables(h, Ws[:, 0::2], Ws[:, 1::2], Wc[:, 0::2], Wc[:, 1::2])

    idxp = jnp.concatenate(
        [idx, jnp.zeros((P - N_IMP, 4), jnp.int32)], axis=0)

    w = _sc_gather_combine(u, v, idxp[:, 0], idxp[:, 1], idxp[:, 2],
                           idxp[:, 3])

    return _mlp(w, b0[0::2].reshape(1, D // 2), b0[1::2].reshape(1, D // 2),
                W1[0::2], W1[1::2], b1.reshape(1, MID), W2,
                b2.reshape(1, MID), W_out, b_out.reshape(1, K_OUT))


# speedup vs baseline: 1.1756x; 1.1756x over previous
import functools

import jax
import jax.numpy as jnp
from jax import lax
from jax.experimental import pallas as pl
from jax.experimental.pallas import tpu as pltpu
from jax.experimental.pallas import tpu_sc as plsc

N_ATOMS = 50000
N_IMP = 100000
D = 128
MID = 128
K_OUT = 6

NC = 2
NS = 16
NW = NC * NS

P = 100352
R_PER_W = P // NW
CHUNK = 112
NCHUNK = R_PER_W // CHUNK


def _sc_gather_body(u_h, v_h, i0_h, i1_h, i2_h, i3_h, w_h,
                    iv, g0a, g0b, g2a, g2b, g3a, g3b, gva, gvb, wb0, wb1,
                    sg0, sg1, sw0, sw1):
    wid = lax.axis_index("s") * NC + lax.axis_index("c")
    base = wid * R_PER_W
    R = R_PER_W
    bufs = ((g0a, g2a, g3a, gva), (g0b, g2b, g3b, gvb))
    wbs = (wb0, wb1)
    sgs = (sg0, sg1)
    sws = (sw0, sw1)

    D2 = D // 2

    pltpu.sync_copy(i0_h.at[pl.ds(base, R)], iv.at[pl.ds(0, R)])
    pltpu.sync_copy(i2_h.at[pl.ds(base, R)], iv.at[pl.ds(R, R)])
    pltpu.sync_copy(i3_h.at[pl.ds(base, R)], iv.at[pl.ds(2 * R, R)])
    pltpu.sync_copy(i1_h.at[pl.ds(base, R)], iv.at[pl.ds(3 * R, R)])

    def fire(kk, b):
        g0, g2, g3, gv = bufs[b]
        pltpu.async_copy(u_h.at[iv.at[pl.ds(kk * CHUNK, CHUNK)]], g0, sgs[b])
        pltpu.async_copy(
            u_h.at[iv.at[pl.ds(R + kk * CHUNK, CHUNK)]], g2, sgs[b])
        pltpu.async_copy(
            u_h.at[iv.at[pl.ds(2 * R + kk * CHUNK, CHUNK)]], g3, sgs[b])
        pltpu.async_copy(
            v_h.at[iv.at[pl.ds(3 * R + kk * CHUNK, CHUNK)]], gv, sgs[b])

    def wait_gathers(b):
        for dst in bufs[b]:
            pltpu.make_async_copy(
                u_h.at[iv.at[pl.ds(0, CHUNK)]], dst, sgs[b]).wait()

    fire(0, 0)

    def outer(k2, carry):
        for b in (0, 1):
            kk = k2 * 2 + b

            @pl.when(kk + 1 < NCHUNK)
            def _():
                fire(kk + 1, 1 - b)

            wait_gathers(b)

            @pl.when(kk >= 2)
            def _():
                pltpu.make_async_copy(
                    wbs[b], w_h.at[pl.ds(base, CHUNK)], sws[b]).wait()

            g0, g2, g3, gv = bufs[b]
            wb = wbs[b]

            def row(r, acc):
                for j in range(D2 // 16):
                    sl = pl.ds(j * 16, 16)
                    a0 = plsc.bitcast(g0[r, sl], jnp.bfloat16)
                    a2 = plsc.bitcast(g2[r, sl], jnp.bfloat16)
                    a3 = plsc.bitcast(g3[r, sl], jnp.bfloat16)
                    av = plsc.bitcast(gv[r, sl], jnp.bfloat16)
                    wb[r, sl] = plsc.bitcast(
                        (a0 + a2) + (a3 + av), jnp.int32)
                return acc

            lax.fori_loop(0, CHUNK, row, 0, unroll=4)
            pltpu.async_copy(
                wb, w_h.at[pl.ds(base + kk * CHUNK, CHUNK)], sws[b])
        return carry

    lax.fori_loop(0, NCHUNK // 2, outer, 0)
    for b in (0, 1):
        pltpu.make_async_copy(
            wbs[b], w_h.at[pl.ds(base, CHUNK)], sws[b]).wait()


def _sc_gather_combine(u, v, i0, i1, i2, i3):
    mesh = plsc.VectorSubcoreMesh(
        core_axis_name="c", subcore_axis_name="s", num_cores=NC, num_subcores=NS
    )
    buf = pltpu.VMEM((CHUNK, D // 2), jnp.int32)
    return pl.kernel(
        _sc_gather_body,
        out_type=jax.ShapeDtypeStruct((P, D // 2), jnp.int32),
        mesh=mesh,
        compiler_params=pltpu.CompilerParams(use_tc_tiling_on_sc=False, needs_layout_passes=False),
        scratch_types=[
            pltpu.VMEM((4 * R_PER_W,), jnp.int32),
            buf, buf, buf, buf, buf, buf, buf, buf, buf, buf,
            pltpu.SemaphoreType.DMA,
            pltpu.SemaphoreType.DMA,
            pltpu.SemaphoreType.DMA,
            pltpu.SemaphoreType.DMA,
        ],
    )(u, v, i0, i1, i2, i3)


_UV_BLK = 2000


def _pack_bf16_pair(lo_f32, hi_f32):
    pe = lax.bitcast_convert_type(
        lo_f32.astype(jnp.bfloat16).astype(jnp.float32), jnp.uint32)
    po = lax.bitcast_convert_type(
        hi_f32.astype(jnp.bfloat16).astype(jnp.float32), jnp.uint32)
    return lax.bitcast_convert_type((pe >> 16) | po, jnp.int32)


def _uv_body(h_ref, w0_ref, b0_ref, u_ref, v_ref):
    ws = (w0_ref[0:D, :] + w0_ref[2 * D:3 * D, :]
          + w0_ref[3 * D:4 * D, :]).astype(jnp.bfloat16)
    wc = (w0_ref[D:2 * D, :] * 3.0).astype(jnp.bfloat16)
    hb = h_ref[:].astype(jnp.bfloat16)
    D2 = D // 2
    ue = jnp.dot(hb, ws[:, :D2], preferred_element_type=jnp.float32)
    uo = jnp.dot(hb, ws[:, D2:], preferred_element_type=jnp.float32)
    ve = (jnp.dot(hb, wc[:, :D2], preferred_element_type=jnp.float32)
          + b0_ref[:, :D2])
    vo = (jnp.dot(hb, wc[:, D2:], preferred_element_type=jnp.float32)
          + b0_ref[:, D2:])
    u_ref[:] = _pack_bf16_pair(ue, uo)
    v_ref[:] = _pack_bf16_pair(ve, vo)


def _uv_tables(h, W0, b0):
    D2 = D // 2
    return pl.pallas_call(
        _uv_body,
        grid=(N_ATOMS // _UV_BLK,),
        in_specs=[
            pl.BlockSpec((_UV_BLK, D), lambda i: (i, 0)),
            pl.BlockSpec((4 * D, D), lambda i: (0, 0)),
            pl.BlockSpec((1, D), lambda i: (0, 0)),
        ],
        out_specs=[
            pl.BlockSpec((_UV_BLK, D2), lambda i: (i, 0)),
            pl.BlockSpec((_UV_BLK, D2), lambda i: (i, 0)),
        ],
        out_shape=[
            jax.ShapeDtypeStruct((N_ATOMS, D2), jnp.int32),
            jax.ShapeDtypeStruct((N_ATOMS, D2), jnp.int32),
        ],
    )(h, W0, b0)


_MLP_BLK = 3584


def _mlp_body(w_ref, w1_ref, b1_ref, w2_ref, b2_ref, wo_ref, bo_ref, out_ref):
    D2 = D // 2
    wi = w_ref[:]
    xe = lax.bitcast_convert_type(wi << 16, jnp.float32)
    xo = lax.bitcast_convert_type(wi & jnp.int32(-65536), jnp.float32)
    xe = jnp.maximum(xe, 0.0)
    xo = jnp.maximum(xo, 0.0)
    x = jnp.maximum(
        jnp.dot(xe, w1_ref[0:D2, :], preferred_element_type=jnp.float32)
        + jnp.dot(xo, w1_ref[D2:, :], preferred_element_type=jnp.float32)
        + b1_ref[:], 0.0)
    x = jnp.maximum(
        jnp.dot(x, w2_ref[:], preferred_element_type=jnp.float32) + b2_ref[:],
        0.0)
    out_ref[:] = (
        jnp.dot(x, wo_ref[:], preferred_element_type=jnp.float32) + bo_ref[:])


def _mlp(w, W1, b1, W2, b2, W_out, b_out):
    kout = W_out.shape[1]
    D2 = D // 2
    return pl.pallas_call(
        _mlp_body,
        grid=(P // _MLP_BLK,),
        in_specs=[
            pl.BlockSpec((_MLP_BLK, D2), lambda i: (i, 0)),
            pl.BlockSpec((MID, MID), lambda i: (0, 0)),
            pl.BlockSpec((1, MID), lambda i: (0, 0)),
            pl.BlockSpec((MID, MID), lambda i: (0, 0)),
            pl.BlockSpec((1, MID), lambda i: (0, 0)),
            pl.BlockSpec((MID, kout), lambda i: (0, 0)),
            pl.BlockSpec((1, kout), lambda i: (0, 0)),
        ],
        out_specs=pl.BlockSpec((_MLP_BLK, kout), lambda i: (i, 0)),
        out_shape=jax.ShapeDtypeStruct((N_IMP, kout), jnp.float32),
    )(w, W1, b1, W2, b2, W_out, b_out)


def kernel(h, idx, W0, b0, W1, b1, W2, b2, W_out, b_out):
    u, v = _uv_tables(h, W0, b0.reshape(1, D))

    idxp = jnp.concatenate(
        [idx, jnp.zeros((P - N_IMP, 4), jnp.int32)], axis=0)

    w = _sc_gather_combine(u, v, idxp[:, 0], idxp[:, 1], idxp[:, 2],
                           idxp[:, 3])

    return _mlp(w, W1, b1.reshape(1, MID), W2, b2.reshape(1, MID),
                W_out, b_out.reshape(1, K_OUT))


# speedup vs baseline: 1.3047x; 1.1098x over previous
import functools

import jax
import jax.numpy as jnp
from jax import lax
from jax.experimental import pallas as pl
from jax.experimental.pallas import tpu as pltpu
from jax.experimental.pallas import tpu_sc as plsc

N_ATOMS = 50000
N_IMP = 100000
D = 128
MID = 128
K_OUT = 6

NC = 2
NS = 16
NW = NC * NS

P = 100352
NSEG = 2
P_SEG = P // NSEG
R_PER_W = P_SEG // NW
CHUNK = 112
NCHUNK = R_PER_W // CHUNK


def _sc_gather_body(u_h, v_h, i0_h, i1_h, i2_h, i3_h, w_h,
                    iv, g0a, g0b, g2a, g2b, g3a, g3b, gva, gvb, wb0, wb1,
                    sg0, sg1, sw0, sw1):
    wid = lax.axis_index("s") * NC + lax.axis_index("c")
    base = wid * R_PER_W
    R = R_PER_W
    bufs = ((g0a, g2a, g3a, gva), (g0b, g2b, g3b, gvb))
    wbs = (wb0, wb1)
    sgs = (sg0, sg1)
    sws = (sw0, sw1)

    D2 = D // 2

    pltpu.sync_copy(i0_h.at[pl.ds(base, R)], iv.at[pl.ds(0, R)])
    pltpu.sync_copy(i2_h.at[pl.ds(base, R)], iv.at[pl.ds(R, R)])
    pltpu.sync_copy(i3_h.at[pl.ds(base, R)], iv.at[pl.ds(2 * R, R)])
    pltpu.sync_copy(i1_h.at[pl.ds(base, R)], iv.at[pl.ds(3 * R, R)])

    def fire(kk, b):
        g0, g2, g3, gv = bufs[b]
        pltpu.async_copy(u_h.at[iv.at[pl.ds(kk * CHUNK, CHUNK)]], g0, sgs[b])
        pltpu.async_copy(
            u_h.at[iv.at[pl.ds(R + kk * CHUNK, CHUNK)]], g2, sgs[b])
        pltpu.async_copy(
            u_h.at[iv.at[pl.ds(2 * R + kk * CHUNK, CHUNK)]], g3, sgs[b])
        pltpu.async_copy(
            v_h.at[iv.at[pl.ds(3 * R + kk * CHUNK, CHUNK)]], gv, sgs[b])

    def wait_gathers(b):
        for dst in bufs[b]:
            pltpu.make_async_copy(
                u_h.at[iv.at[pl.ds(0, CHUNK)]], dst, sgs[b]).wait()

    fire(0, 0)

    def outer(k2, carry):
        for b in (0, 1):
            kk = k2 * 2 + b

            @pl.when(kk + 1 < NCHUNK)
            def _():
                fire(kk + 1, 1 - b)

            wait_gathers(b)

            @pl.when(kk >= 2)
            def _():
                pltpu.make_async_copy(
                    wbs[b], w_h.at[pl.ds(base, CHUNK)], sws[b]).wait()

            g0, g2, g3, gv = bufs[b]
            wb = wbs[b]

            def row(r, acc):
                for j in range(D2 // 16):
                    sl = pl.ds(j * 16, 16)
                    a0 = plsc.bitcast(g0[r, sl], jnp.bfloat16)
                    a2 = plsc.bitcast(g2[r, sl], jnp.bfloat16)
                    a3 = plsc.bitcast(g3[r, sl], jnp.bfloat16)
                    av = plsc.bitcast(gv[r, sl], jnp.bfloat16)
                    wb[r, sl] = plsc.bitcast(
                        (a0 + a2) + (a3 + av), jnp.int32)
                return acc

            lax.fori_loop(0, CHUNK, row, 0, unroll=4)
            pltpu.async_copy(
                wb, w_h.at[pl.ds(base + kk * CHUNK, CHUNK)], sws[b])
        return carry

    lax.fori_loop(0, NCHUNK // 2, outer, 0)
    for b in (0, 1):
        pltpu.make_async_copy(
            wbs[b], w_h.at[pl.ds(base, CHUNK)], sws[b]).wait()


def _sc_gather_combine(u, v, i0, i1, i2, i3):
    mesh = plsc.VectorSubcoreMesh(
        core_axis_name="c", subcore_axis_name="s", num_cores=NC, num_subcores=NS
    )
    buf = pltpu.VMEM((CHUNK, D // 2), jnp.int32)
    return pl.kernel(
        _sc_gather_body,
        out_type=jax.ShapeDtypeStruct((P_SEG, D // 2), jnp.int32),
        mesh=mesh,
        compiler_params=pltpu.CompilerParams(use_tc_tiling_on_sc=False, needs_layout_passes=False),
        scratch_types=[
            pltpu.VMEM((4 * R_PER_W,), jnp.int32),
            buf, buf, buf, buf, buf, buf, buf, buf, buf, buf,
            pltpu.SemaphoreType.DMA,
            pltpu.SemaphoreType.DMA,
            pltpu.SemaphoreType.DMA,
            pltpu.SemaphoreType.DMA,
        ],
    )(u, v, i0, i1, i2, i3)


_UV_BLK = 2000


def _pack_bf16_pair(lo_f32, hi_f32):
    pe = lax.bitcast_convert_type(
        lo_f32.astype(jnp.bfloat16).astype(jnp.float32), jnp.uint32)
    po = lax.bitcast_convert_type(
        hi_f32.astype(jnp.bfloat16).astype(jnp.float32), jnp.uint32)
    return lax.bitcast_convert_type((pe >> 16) | po, jnp.int32)


def _uv_body(h_ref, w0_ref, b0_ref, u_ref, v_ref):
    ws = (w0_ref[0:D, :] + w0_ref[2 * D:3 * D, :]
          + w0_ref[3 * D:4 * D, :]).astype(jnp.bfloat16)
    wc = (w0_ref[D:2 * D, :] * 3.0).astype(jnp.bfloat16)
    hb = h_ref[:].astype(jnp.bfloat16)
    D2 = D // 2
    ue = jnp.dot(hb, ws[:, :D2], preferred_element_type=jnp.float32)
    uo = jnp.dot(hb, ws[:, D2:], preferred_element_type=jnp.float32)
    ve = (jnp.dot(hb, wc[:, :D2], preferred_element_type=jnp.float32)
          + b0_ref[:, :D2])
    vo = (jnp.dot(hb, wc[:, D2:], preferred_element_type=jnp.float32)
          + b0_ref[:, D2:])
    u_ref[:] = _pack_bf16_pair(ue, uo)
    v_ref[:] = _pack_bf16_pair(ve, vo)


def _uv_tables(h, W0, b0):
    D2 = D // 2
    return pl.pallas_call(
        _uv_body,
        grid=(N_ATOMS // _UV_BLK,),
        in_specs=[
            pl.BlockSpec((_UV_BLK, D), lambda i: (i, 0)),
            pl.BlockSpec((4 * D, D), lambda i: (0, 0)),
            pl.BlockSpec((1, D), lambda i: (0, 0)),
        ],
        out_specs=[
            pl.BlockSpec((_UV_BLK, D2), lambda i: (i, 0)),
            pl.BlockSpec((_UV_BLK, D2), lambda i: (i, 0)),
        ],
        out_shape=[
            jax.ShapeDtypeStruct((N_ATOMS, D2), jnp.int32),
            jax.ShapeDtypeStruct((N_ATOMS, D2), jnp.int32),
        ],
    )(h, W0, b0)


_MLP_BLK = 3584


def _mlp_body(w_ref, w1_ref, b1_ref, w2_ref, b2_ref, wo_ref, bo_ref, out_ref):
    D2 = D // 2
    wi = w_ref[:]
    xe = lax.bitcast_convert_type(wi << 16, jnp.float32)
    xo = lax.bitcast_convert_type(wi & jnp.int32(-65536), jnp.float32)
    xe = jnp.maximum(xe, 0.0)
    xo = jnp.maximum(xo, 0.0)
    x = jnp.maximum(
        jnp.dot(xe, w1_ref[0:D2, :], preferred_element_type=jnp.float32)
        + jnp.dot(xo, w1_ref[D2:, :], preferred_element_type=jnp.float32)
        + b1_ref[:], 0.0)
    x = jnp.maximum(
        jnp.dot(x, w2_ref[:], preferred_element_type=jnp.float32) + b2_ref[:],
        0.0)
    out_ref[:] = (
        jnp.dot(x, wo_ref[:], preferred_element_type=jnp.float32) + bo_ref[:])


def _mlp(w, W1, b1, W2, b2, W_out, b_out, n_valid):
    kout = W_out.shape[1]
    D2 = D // 2
    return pl.pallas_call(
        _mlp_body,
        grid=(P_SEG // _MLP_BLK,),
        in_specs=[
            pl.BlockSpec((_MLP_BLK, D2), lambda i: (i, 0)),
            pl.BlockSpec((MID, MID), lambda i: (0, 0)),
            pl.BlockSpec((1, MID), lambda i: (0, 0)),
            pl.BlockSpec((MID, MID), lambda i: (0, 0)),
            pl.BlockSpec((1, MID), lambda i: (0, 0)),
            pl.BlockSpec((MID, kout), lambda i: (0, 0)),
            pl.BlockSpec((1, kout), lambda i: (0, 0)),
        ],
        out_specs=pl.BlockSpec((_MLP_BLK, kout), lambda i: (i, 0)),
        out_shape=jax.ShapeDtypeStruct((n_valid, kout), jnp.float32),
    )(w, W1, b1, W2, b2, W_out, b_out)


def kernel(h, idx, W0, b0, W1, b1, W2, b2, W_out, b_out):
    u, v = _uv_tables(h, W0, b0.reshape(1, D))

    idxp = jnp.concatenate(
        [idx, jnp.zeros((P - N_IMP, 4), jnp.int32)], axis=0)
    i0, i1, i2, i3 = idxp[:, 0], idxp[:, 1], idxp[:, 2], idxp[:, 3]

    b1r = b1.reshape(1, MID)
    b2r = b2.reshape(1, MID)
    bor = b_out.reshape(1, K_OUT)

    outs = []
    for s in range(NSEG):
        lo = s * P_SEG
        sl = slice(lo, lo + P_SEG)
        w = _sc_gather_combine(u, v, i0[sl], i1[sl], i2[sl], i3[sl])
        n_valid = min(P_SEG, N_IMP - lo)
        outs.append(_mlp(w, W1, b1r, W2, b2r, W_out, bor, n_valid))
    return jnp.concatenate(outs, axis=0)


# speedup vs baseline: 1.4614x; 1.1201x over previous
import functools

import jax
import jax.numpy as jnp
from jax import lax
from jax.experimental import pallas as pl
from jax.experimental.pallas import tpu as pltpu
from jax.experimental.pallas import tpu_sc as plsc

N_ATOMS = 50000
N_IMP = 100000
D = 128
MID = 128
K_OUT = 6

NC = 2
NS = 16
NW = NC * NS

P = 100352
NSEG = 2
P_SEG = P // NSEG
R_PER_W = P_SEG // NW
CHUNK = 112
NCHUNK = R_PER_W // CHUNK


def _sc_gather_body(u_h, v_h, i0_h, i1_h, i2_h, i3_h, w_h,
                    iv, g0a, g0b, g2a, g2b, g3a, g3b, gva, gvb, wb0, wb1,
                    sg0, sg1, sw0, sw1):
    wid = lax.axis_index("s") * NC + lax.axis_index("c")
    base = wid * R_PER_W
    R = R_PER_W
    bufs = ((g0a, g2a, g3a, gva), (g0b, g2b, g3b, gvb))
    wbs = (wb0, wb1)
    sgs = (sg0, sg1)
    sws = (sw0, sw1)

    D2 = D // 2

    pltpu.sync_copy(i0_h.at[pl.ds(base, R)], iv.at[pl.ds(0, R)])
    pltpu.sync_copy(i2_h.at[pl.ds(base, R)], iv.at[pl.ds(R, R)])
    pltpu.sync_copy(i3_h.at[pl.ds(base, R)], iv.at[pl.ds(2 * R, R)])
    pltpu.sync_copy(i1_h.at[pl.ds(base, R)], iv.at[pl.ds(3 * R, R)])

    def fire(kk, b):
        g0, g2, g3, gv = bufs[b]
        pltpu.async_copy(u_h.at[iv.at[pl.ds(kk * CHUNK, CHUNK)]], g0, sgs[b])
        pltpu.async_copy(
            u_h.at[iv.at[pl.ds(R + kk * CHUNK, CHUNK)]], g2, sgs[b])
        pltpu.async_copy(
            u_h.at[iv.at[pl.ds(2 * R + kk * CHUNK, CHUNK)]], g3, sgs[b])
        pltpu.async_copy(
            v_h.at[iv.at[pl.ds(3 * R + kk * CHUNK, CHUNK)]], gv, sgs[b])

    def wait_gathers(b):
        for dst in bufs[b]:
            pltpu.make_async_copy(
                u_h.at[iv.at[pl.ds(0, CHUNK)]], dst, sgs[b]).wait()

    fire(0, 0)

    def outer(k2, carry):
        for b in (0, 1):
            kk = k2 * 2 + b

            @pl.when(kk + 1 < NCHUNK)
            def _():
                fire(kk + 1, 1 - b)

            wait_gathers(b)

            @pl.when(kk >= 2)
            def _():
                pltpu.make_async_copy(
                    wbs[b], w_h.at[pl.ds(base, CHUNK)], sws[b]).wait()

            g0, g2, g3, gv = bufs[b]
            wb = wbs[b]

            def row(r, acc):
                for j in range(D2 // 16):
                    sl = pl.ds(j * 16, 16)
                    a0 = plsc.bitcast(g0[r, sl], jnp.bfloat16)
                    a2 = plsc.bitcast(g2[r, sl], jnp.bfloat16)
                    a3 = plsc.bitcast(g3[r, sl], jnp.bfloat16)
                    av = plsc.bitcast(gv[r, sl], jnp.bfloat16)
                    wb[r, sl] = plsc.bitcast(
                        (a0 + a2) + (a3 + av), jnp.int32)
                return acc

            lax.fori_loop(0, CHUNK, row, 0, unroll=4)
            pltpu.async_copy(
                wb, w_h.at[pl.ds(base + kk * CHUNK, CHUNK)], sws[b])
        return carry

    lax.fori_loop(0, NCHUNK // 2, outer, 0)
    for b in (0, 1):
        pltpu.make_async_copy(
            wbs[b], w_h.at[pl.ds(base, CHUNK)], sws[b]).wait()


def _sc_gather_combine(u, v, i0, i1, i2, i3):
    mesh = plsc.VectorSubcoreMesh(
        core_axis_name="c", subcore_axis_name="s", num_cores=NC, num_subcores=NS
    )
    buf = pltpu.VMEM((CHUNK, D // 2), jnp.int32)
    return pl.kernel(
        _sc_gather_body,
        out_type=jax.ShapeDtypeStruct((P_SEG, D // 2), jnp.int32),
        mesh=mesh,
        compiler_params=pltpu.CompilerParams(use_tc_tiling_on_sc=False, needs_layout_passes=False),
        scratch_types=[
            pltpu.VMEM((4 * R_PER_W,), jnp.int32),
            buf, buf, buf, buf, buf, buf, buf, buf, buf, buf,
            pltpu.SemaphoreType.DMA,
            pltpu.SemaphoreType.DMA,
            pltpu.SemaphoreType.DMA,
            pltpu.SemaphoreType.DMA,
        ],
    )(u, v, i0, i1, i2, i3)


_UV_BLK = 2000


def _pack_bf16_pair(lo_f32, hi_f32):
    pe = lax.bitcast_convert_type(
        lo_f32.astype(jnp.bfloat16).astype(jnp.float32), jnp.uint32)
    po = lax.bitcast_convert_type(
        hi_f32.astype(jnp.bfloat16).astype(jnp.float32), jnp.uint32)
    return lax.bitcast_convert_type((pe >> 16) | po, jnp.int32)


def _uv_body(h_ref, w0_ref, b0_ref, u_ref, v_ref):
    ws = (w0_ref[0:D, :] + w0_ref[2 * D:3 * D, :]
          + w0_ref[3 * D:4 * D, :]).astype(jnp.bfloat16)
    wc = (w0_ref[D:2 * D, :] * 3.0).astype(jnp.bfloat16)
    hb = h_ref[:].astype(jnp.bfloat16)
    D2 = D // 2
    ue = jnp.dot(hb, ws[:, :D2], preferred_element_type=jnp.float32)
    uo = jnp.dot(hb, ws[:, D2:], preferred_element_type=jnp.float32)
    ve = (jnp.dot(hb, wc[:, :D2], preferred_element_type=jnp.float32)
          + b0_ref[:, :D2])
    vo = (jnp.dot(hb, wc[:, D2:], preferred_element_type=jnp.float32)
          + b0_ref[:, D2:])
    u_ref[:] = _pack_bf16_pair(ue, uo)
    v_ref[:] = _pack_bf16_pair(ve, vo)


def _uv_tables(h, W0, b0):
    D2 = D // 2
    return pl.pallas_call(
        _uv_body,
        grid=(N_ATOMS // _UV_BLK,),
        in_specs=[
            pl.BlockSpec((_UV_BLK, D), lambda i: (i, 0)),
            pl.BlockSpec((4 * D, D), lambda i: (0, 0)),
            pl.BlockSpec((1, D), lambda i: (0, 0)),
        ],
        out_specs=[
            pl.BlockSpec((_UV_BLK, D2), lambda i: (i, 0)),
            pl.BlockSpec((_UV_BLK, D2), lambda i: (i, 0)),
        ],
        out_shape=[
            jax.ShapeDtypeStruct((N_ATOMS, D2), jnp.int32),
            jax.ShapeDtypeStruct((N_ATOMS, D2), jnp.int32),
        ],
    )(h, W0, b0)


_MLP_BLK = 3584


def _mlp_body(w_ref, w1_ref, b1_ref, w2_ref, b2_ref, wo_ref, bo_ref, out_ref):
    D2 = D // 2
    B2 = _MLP_BLK // 2
    wi = w_ref[:].reshape(B2, D)
    xl = jnp.maximum(lax.bitcast_convert_type(wi << 16, jnp.float32), 0.0)
    xh = jnp.maximum(
        lax.bitcast_convert_type(wi & jnp.int32(-65536), jnp.float32), 0.0)
    w1lo = w1_ref[0:D2, :]
    w1hi = w1_ref[D2:, :]
    z = jnp.zeros((D2, MID), jnp.float32)
    wa = jnp.concatenate(
        [jnp.concatenate([w1lo, z], axis=1),
         jnp.concatenate([z, w1lo], axis=1)], axis=0)
    wb = jnp.concatenate(
        [jnp.concatenate([w1hi, z], axis=1),
         jnp.concatenate([z, w1hi], axis=1)], axis=0)
    b1d = jnp.concatenate([b1_ref[:], b1_ref[:]], axis=1)
    y = (jnp.dot(xl, wa, preferred_element_type=jnp.float32)
         + jnp.dot(xh, wb, preferred_element_type=jnp.float32) + b1d)
    x = jnp.maximum(y.reshape(_MLP_BLK, MID), 0.0)
    x = jnp.maximum(
        jnp.dot(x, w2_ref[:], preferred_element_type=jnp.float32) + b2_ref[:],
        0.0)
    out_ref[:] = (
        jnp.dot(x, wo_ref[:], preferred_element_type=jnp.float32) + bo_ref[:])


def _mlp(w, W1, b1, W2, b2, W_out, b_out, n_valid):
    kout = W_out.shape[1]
    return pl.pallas_call(
        _mlp_body,
        grid=(P_SEG // _MLP_BLK,),
        in_specs=[
            pl.BlockSpec((_MLP_BLK // 16, 8, D), lambda i: (i, 0, 0)),
            pl.BlockSpec((MID, MID), lambda i: (0, 0)),
            pl.BlockSpec((1, MID), lambda i: (0, 0)),
            pl.BlockSpec((MID, MID), lambda i: (0, 0)),
            pl.BlockSpec((1, MID), lambda i: (0, 0)),
            pl.BlockSpec((MID, kout), lambda i: (0, 0)),
            pl.BlockSpec((1, kout), lambda i: (0, 0)),
        ],
        out_specs=pl.BlockSpec((_MLP_BLK, kout), lambda i: (i, 0)),
        out_shape=jax.ShapeDtypeStruct((n_valid, kout), jnp.float32),
    )(w, W1, b1, W2, b2, W_out, b_out)


def kernel(h, idx, W0, b0, W1, b1, W2, b2, W_out, b_out):
    u, v = _uv_tables(h, W0, b0.reshape(1, D))

    idxp = jnp.concatenate(
        [idx, jnp.zeros((P - N_IMP, 4), jnp.int32)], axis=0)
    i0, i1, i2, i3 = idxp[:, 0], idxp[:, 1], idxp[:, 2], idxp[:, 3]

    b1r = b1.reshape(1, MID)
    b2r = b2.reshape(1, MID)
    bor = b_out.reshape(1, K_OUT)

    outs = []
    for s in range(NSEG):
        lo = s * P_SEG
        sl = slice(lo, lo + P_SEG)
        w = _sc_gather_combine(u, v, i0[sl], i1[sl], i2[sl], i3[sl])
        w3 = w.reshape(P_SEG // 16, 8, D)
        n_valid = min(P_SEG, N_IMP - lo)
        outs.append(_mlp(w3, W1, b1r, W2, b2r, W_out, bor, n_valid))
    return jnp.concatenate(outs, axis=0)


# speedup vs baseline: 1.4640x; 1.0018x over previous
import functools

import jax
import jax.numpy as jnp
from jax import lax
from jax.experimental import pallas as pl
from jax.experimental.pallas import tpu as pltpu
from jax.experimental.pallas import tpu_sc as plsc

N_ATOMS = 50000
N_IMP = 100000
D = 128
MID = 128
K_OUT = 6

NC = 2
NS = 16
NW = NC * NS

P = 100352
NSEG = 2
P_SEG = P // NSEG
R_PER_W = P_SEG // NW
CHUNK = 112
NCHUNK = R_PER_W // CHUNK


def _sc_gather_body(u_h, v_h, i0_h, i1_h, i2_h, i3_h, w_h,
                    iv, g0a, g0b, g2a, g2b, g3a, g3b, gva, gvb, wb0, wb1,
                    sg0, sg1, sw0, sw1):
    wid = lax.axis_index("s") * NC + lax.axis_index("c")
    base = wid * R_PER_W
    R = R_PER_W
    bufs = ((g0a, g2a, g3a, gva), (g0b, g2b, g3b, gvb))
    wbs = (wb0, wb1)
    sgs = (sg0, sg1)
    sws = (sw0, sw1)

    D2 = D // 2

    pltpu.sync_copy(i0_h.at[pl.ds(base, R)], iv.at[pl.ds(0, R)])
    pltpu.sync_copy(i2_h.at[pl.ds(base, R)], iv.at[pl.ds(R, R)])
    pltpu.sync_copy(i3_h.at[pl.ds(base, R)], iv.at[pl.ds(2 * R, R)])
    pltpu.sync_copy(i1_h.at[pl.ds(base, R)], iv.at[pl.ds(3 * R, R)])

    def fire(kk, b):
        g0, g2, g3, gv = bufs[b]
        pltpu.async_copy(u_h.at[iv.at[pl.ds(kk * CHUNK, CHUNK)]], g0, sgs[b])
        pltpu.async_copy(
            u_h.at[iv.at[pl.ds(R + kk * CHUNK, CHUNK)]], g2, sgs[b])
        pltpu.async_copy(
            u_h.at[iv.at[pl.ds(2 * R + kk * CHUNK, CHUNK)]], g3, sgs[b])
        pltpu.async_copy(
            v_h.at[iv.at[pl.ds(3 * R + kk * CHUNK, CHUNK)]], gv, sgs[b])

    def wait_gathers(b):
        for dst in bufs[b]:
            pltpu.make_async_copy(
                u_h.at[iv.at[pl.ds(0, CHUNK)]], dst, sgs[b]).wait()

    fire(0, 0)

    def outer(k2, carry):
        for b in (0, 1):
            kk = k2 * 2 + b

            @pl.when(kk + 1 < NCHUNK)
            def _():
                fire(kk + 1, 1 - b)

            wait_gathers(b)

            @pl.when(kk >= 2)
            def _():
                pltpu.make_async_copy(
                    wbs[b], w_h.at[pl.ds(base, CHUNK)], sws[b]).wait()

            g0, g2, g3, gv = bufs[b]
            wb = wbs[b]

            def row(r, acc):
                for j in range(D2 // 16):
                    sl = pl.ds(j * 16, 16)
                    a0 = plsc.bitcast(g0[r, sl], jnp.bfloat16)
                    a2 = plsc.bitcast(g2[r, sl], jnp.bfloat16)
                    a3 = plsc.bitcast(g3[r, sl], jnp.bfloat16)
                    av = plsc.bitcast(gv[r, sl], jnp.bfloat16)
                    wb[r, sl] = plsc.bitcast(
                        (a0 + a2) + (a3 + av), jnp.int32)
                return acc

            lax.fori_loop(0, CHUNK, row, 0, unroll=4)
            pltpu.async_copy(
                wb, w_h.at[pl.ds(base + kk * CHUNK, CHUNK)], sws[b])
        return carry

    lax.fori_loop(0, NCHUNK // 2, outer, 0)
    for b in (0, 1):
        pltpu.make_async_copy(
            wbs[b], w_h.at[pl.ds(base, CHUNK)], sws[b]).wait()


def _sc_gather_combine(u, v, i0, i1, i2, i3):
    mesh = plsc.VectorSubcoreMesh(
        core_axis_name="c", subcore_axis_name="s", num_cores=NC, num_subcores=NS
    )
    buf = pltpu.VMEM((CHUNK, D // 2), jnp.int32)
    return pl.kernel(
        _sc_gather_body,
        out_type=jax.ShapeDtypeStruct((P_SEG, D // 2), jnp.int32),
        mesh=mesh,
        compiler_params=pltpu.CompilerParams(use_tc_tiling_on_sc=False, needs_layout_passes=False),
        scratch_types=[
            pltpu.VMEM((4 * R_PER_W,), jnp.int32),
            buf, buf, buf, buf, buf, buf, buf, buf, buf, buf,
            pltpu.SemaphoreType.DMA,
            pltpu.SemaphoreType.DMA,
            pltpu.SemaphoreType.DMA,
            pltpu.SemaphoreType.DMA,
        ],
    )(u, v, i0, i1, i2, i3)


_UV_BLK = 2000


def _pack_bf16_pair(lo_f32, hi_f32):
    pe = lax.bitcast_convert_type(
        lo_f32.astype(jnp.bfloat16).astype(jnp.float32), jnp.uint32)
    po = lax.bitcast_convert_type(
        hi_f32.astype(jnp.bfloat16).astype(jnp.float32), jnp.uint32)
    return lax.bitcast_convert_type((pe >> 16) | po, jnp.int32)


def _uv_body(h_ref, w0_ref, b0_ref, u_ref, v_ref):
    ws = (w0_ref[0:D, :] + w0_ref[2 * D:3 * D, :]
          + w0_ref[3 * D:4 * D, :]).astype(jnp.bfloat16)
    wc = (w0_ref[D:2 * D, :] * 3.0).astype(jnp.bfloat16)
    hb = h_ref[:].astype(jnp.bfloat16)
    D2 = D // 2
    ue = jnp.dot(hb, ws[:, :D2], preferred_element_type=jnp.float32)
    uo = jnp.dot(hb, ws[:, D2:], preferred_element_type=jnp.float32)
    ve = (jnp.dot(hb, wc[:, :D2], preferred_element_type=jnp.float32)
          + b0_ref[:, :D2])
    vo = (jnp.dot(hb, wc[:, D2:], preferred_element_type=jnp.float32)
          + b0_ref[:, D2:])
    u_ref[:] = _pack_bf16_pair(ue, uo)
    v_ref[:] = _pack_bf16_pair(ve, vo)


def _uv_tables(h, W0, b0):
    return pl.pallas_call(
        _uv_body,
        grid=(N_ATOMS // _UV_BLK,),
        in_specs=[
            pl.BlockSpec((_UV_BLK, D), lambda i: (i, 0)),
            pl.BlockSpec((4 * D, D), lambda i: (0, 0)),
            pl.BlockSpec((1, D), lambda i: (0, 0)),
        ],
        out_specs=[
            pl.BlockSpec((_UV_BLK, D // 2), lambda i: (i, 0)),
            pl.BlockSpec((_UV_BLK, D // 2), lambda i: (i, 0)),
        ],
        out_shape=[
            jax.ShapeDtypeStruct((N_ATOMS, D // 2), jnp.int32),
            jax.ShapeDtypeStruct((N_ATOMS, D // 2), jnp.int32),
        ],
    )(h, W0, b0)


_MLP_BLK = 3584


def _mlp_body(w_ref, w1_ref, b1_ref, w2_ref, b2_ref, wo_ref, bo_ref, out_ref):
    D2 = D // 2
    B2 = _MLP_BLK // 2
    wi = w_ref[:].reshape(B2, D)
    xl = jnp.maximum(lax.bitcast_convert_type(wi << 16, jnp.float32), 0.0)
    xh = jnp.maximum(
        lax.bitcast_convert_type(wi & jnp.int32(-65536), jnp.float32), 0.0)
    w1lo = w1_ref[0:D2, :]
    w1hi = w1_ref[D2:, :]
    z = jnp.zeros((D2, MID), jnp.float32)
    wa = jnp.concatenate(
        [jnp.concatenate([w1lo, z], axis=1),
         jnp.concatenate([z, w1lo], axis=1)], axis=0)
    wb = jnp.concatenate(
        [jnp.concatenate([w1hi, z], axis=1),
         jnp.concatenate([z, w1hi], axis=1)], axis=0)
    b1d = jnp.concatenate([b1_ref[:], b1_ref[:]], axis=1)
    y = (jnp.dot(xl, wa, preferred_element_type=jnp.float32)
         + jnp.dot(xh, wb, preferred_element_type=jnp.float32) + b1d)
    x = jnp.maximum(y.reshape(_MLP_BLK, MID), 0.0)
    x = jnp.maximum(
        jnp.dot(x, w2_ref[:], preferred_element_type=jnp.float32) + b2_ref[:],
        0.0)
    out_ref[:] = (
        jnp.dot(x, wo_ref[:], preferred_element_type=jnp.float32) + bo_ref[:])


def _mlp(w, W1, b1, W2, b2, W_out, b_out, n_valid):
    kout = W_out.shape[1]
    return pl.pallas_call(
        _mlp_body,
        grid=(P_SEG // _MLP_BLK,),
        in_specs=[
            pl.BlockSpec((_MLP_BLK // 16, 8, D), lambda i: (i, 0, 0)),
            pl.BlockSpec((MID, MID), lambda i: (0, 0)),
            pl.BlockSpec((1, MID), lambda i: (0, 0)),
            pl.BlockSpec((MID, MID), lambda i: (0, 0)),
            pl.BlockSpec((1, MID), lambda i: (0, 0)),
            pl.BlockSpec((MID, kout), lambda i: (0, 0)),
            pl.BlockSpec((1, kout), lambda i: (0, 0)),
        ],
        out_specs=pl.BlockSpec((_MLP_BLK, kout), lambda i: (i, 0)),
        out_shape=jax.ShapeDtypeStruct((n_valid, kout), jnp.float32),
    )(w, W1, b1, W2, b2, W_out, b_out)


def kernel(h, idx, W0, b0, W1, b1, W2, b2, W_out, b_out):
    u, v = _uv_tables(h, W0, b0.reshape(1, D))

    idxp = jnp.concatenate(
        [idx, jnp.zeros((P - N_IMP, 4), jnp.int32)], axis=0)
    i0, i1, i2, i3 = idxp[:, 0], idxp[:, 1], idxp[:, 2], idxp[:, 3]

    b1r = b1.reshape(1, MID)
    b2r = b2.reshape(1, MID)
    bor = b_out.reshape(1, K_OUT)

    outs = []
    for s in range(NSEG):
        lo = s * P_SEG
        sl = slice(lo, lo + P_SEG)
        w = _sc_gather_combine(u, v, i0[sl], i1[sl], i2[sl], i3[sl])
        w3 = w.reshape(P_SEG // 16, 8, D)
        n_valid = min(P_SEG, N_IMP - lo)
        outs.append(_mlp(w3, W1, b1r, W2, b2r, W_out, bor, n_valid))
    return jnp.concatenate(outs, axis=0)


# speedup vs baseline: 1.4671x; 1.0021x over previous
import functools

import jax
import jax.numpy as jnp
from jax import lax
from jax.experimental import pallas as pl
from jax.experimental.pallas import tpu as pltpu
from jax.experimental.pallas import tpu_sc as plsc

N_ATOMS = 50000
N_IMP = 100000
D = 128
MID = 128
K_OUT = 6

NC = 2
NS = 16
NW = NC * NS

P = 100352
NSEG = 4
P_SEG = P // NSEG
R_PER_W = P_SEG // NW
CHUNK = 56
NCHUNK = R_PER_W // CHUNK


def _sc_gather_body(u_h, v_h, i0_h, i1_h, i2_h, i3_h, w_h,
                    iv, g0a, g0b, g2a, g2b, g3a, g3b, gva, gvb, wb0, wb1,
                    sg0, sg1, sw0, sw1):
    wid = lax.axis_index("s") * NC + lax.axis_index("c")
    base = wid * R_PER_W
    R = R_PER_W
    bufs = ((g0a, g2a, g3a, gva), (g0b, g2b, g3b, gvb))
    wbs = (wb0, wb1)
    sgs = (sg0, sg1)
    sws = (sw0, sw1)

    D2 = D // 2

    pltpu.sync_copy(i0_h.at[pl.ds(base, R)], iv.at[pl.ds(0, R)])
    pltpu.sync_copy(i2_h.at[pl.ds(base, R)], iv.at[pl.ds(R, R)])
    pltpu.sync_copy(i3_h.at[pl.ds(base, R)], iv.at[pl.ds(2 * R, R)])
    pltpu.sync_copy(i1_h.at[pl.ds(base, R)], iv.at[pl.ds(3 * R, R)])

    def fire(kk, b):
        g0, g2, g3, gv = bufs[b]
        pltpu.async_copy(u_h.at[iv.at[pl.ds(kk * CHUNK, CHUNK)]], g0, sgs[b])
        pltpu.async_copy(
            u_h.at[iv.at[pl.ds(R + kk * CHUNK, CHUNK)]], g2, sgs[b])
        pltpu.async_copy(
            u_h.at[iv.at[pl.ds(2 * R + kk * CHUNK, CHUNK)]], g3, sgs[b])
        pltpu.async_copy(
            v_h.at[iv.at[pl.ds(3 * R + kk * CHUNK, CHUNK)]], gv, sgs[b])

    def wait_gathers(b):
        for dst in bufs[b]:
            pltpu.make_async_copy(
                u_h.at[iv.at[pl.ds(0, CHUNK)]], dst, sgs[b]).wait()

    fire(0, 0)

    def outer(k2, carry):
        for b in (0, 1):
            kk = k2 * 2 + b

            @pl.when(kk + 1 < NCHUNK)
            def _():
                fire(kk + 1, 1 - b)

            wait_gathers(b)

            @pl.when(kk >= 2)
            def _():
                pltpu.make_async_copy(
                    wbs[b], w_h.at[pl.ds(base, CHUNK)], sws[b]).wait()

            g0, g2, g3, gv = bufs[b]
            wb = wbs[b]

            def row(r, acc):
                for j in range(D2 // 16):
                    sl = pl.ds(j * 16, 16)
                    a0 = plsc.bitcast(g0[r, sl], jnp.bfloat16)
                    a2 = plsc.bitcast(g2[r, sl], jnp.bfloat16)
                    a3 = plsc.bitcast(g3[r, sl], jnp.bfloat16)
                    av = plsc.bitcast(gv[r, sl], jnp.bfloat16)
                    wb[r, sl] = plsc.bitcast(
                        (a0 + a2) + (a3 + av), jnp.int32)
                return acc

            lax.fori_loop(0, CHUNK, row, 0, unroll=4)
            pltpu.async_copy(
                wb, w_h.at[pl.ds(base + kk * CHUNK, CHUNK)], sws[b])
        return carry

    lax.fori_loop(0, NCHUNK // 2, outer, 0)
    for b in (0, 1):
        pltpu.make_async_copy(
            wbs[b], w_h.at[pl.ds(base, CHUNK)], sws[b]).wait()


def _sc_gather_combine(u, v, i0, i1, i2, i3):
    mesh = plsc.VectorSubcoreMesh(
        core_axis_name="c", subcore_axis_name="s", num_cores=NC, num_subcores=NS
    )
    buf = pltpu.VMEM((CHUNK, D // 2), jnp.int32)
    return pl.kernel(
        _sc_gather_body,
        out_type=jax.ShapeDtypeStruct((P_SEG, D // 2), jnp.int32),
        mesh=mesh,
        compiler_params=pltpu.CompilerParams(use_tc_tiling_on_sc=False, needs_layout_passes=False),
        scratch_types=[
            pltpu.VMEM((4 * R_PER_W,), jnp.int32),
            buf, buf, buf, buf, buf, buf, buf, buf, buf, buf,
            pltpu.SemaphoreType.DMA,
            pltpu.SemaphoreType.DMA,
            pltpu.SemaphoreType.DMA,
            pltpu.SemaphoreType.DMA,
        ],
    )(u, v, i0, i1, i2, i3)


_UV_BLK = 2000


def _pack_bf16_pair(lo_f32, hi_f32):
    pe = lax.bitcast_convert_type(
        lo_f32.astype(jnp.bfloat16).astype(jnp.float32), jnp.uint32)
    po = lax.bitcast_convert_type(
        hi_f32.astype(jnp.bfloat16).astype(jnp.float32), jnp.uint32)
    return lax.bitcast_convert_type((pe >> 16) | po, jnp.int32)


def _uv_body(h_ref, w0_ref, b0_ref, u_ref, v_ref):
    ws = (w0_ref[0:D, :] + w0_ref[2 * D:3 * D, :]
          + w0_ref[3 * D:4 * D, :]).astype(jnp.bfloat16)
    wc = (w0_ref[D:2 * D, :] * 3.0).astype(jnp.bfloat16)
    hb = h_ref[:].astype(jnp.bfloat16)
    D2 = D // 2
    ue = jnp.dot(hb, ws[:, :D2], preferred_element_type=jnp.float32)
    uo = jnp.dot(hb, ws[:, D2:], preferred_element_type=jnp.float32)
    ve = (jnp.dot(hb, wc[:, :D2], preferred_element_type=jnp.float32)
          + b0_ref[:, :D2])
    vo = (jnp.dot(hb, wc[:, D2:], preferred_element_type=jnp.float32)
          + b0_ref[:, D2:])
    u_ref[:] = _pack_bf16_pair(ue, uo)
    v_ref[:] = _pack_bf16_pair(ve, vo)


def _uv_tables(h, W0, b0):
    return pl.pallas_call(
        _uv_body,
        grid=(N_ATOMS // _UV_BLK,),
        in_specs=[
            pl.BlockSpec((_UV_BLK, D), lambda i: (i, 0)),
            pl.BlockSpec((4 * D, D), lambda i: (0, 0)),
            pl.BlockSpec((1, D), lambda i: (0, 0)),
        ],
        out_specs=[
            pl.BlockSpec((_UV_BLK, D // 2), lambda i: (i, 0)),
            pl.BlockSpec((_UV_BLK, D // 2), lambda i: (i, 0)),
        ],
        out_shape=[
            jax.ShapeDtypeStruct((N_ATOMS, D // 2), jnp.int32),
            jax.ShapeDtypeStruct((N_ATOMS, D // 2), jnp.int32),
        ],
    )(h, W0, b0)


_MLP_BLK = 3584


def _mlp_body(w_ref, w1_ref, b1_ref, w2_ref, b2_ref, wo_ref, bo_ref, out_ref):
    D2 = D // 2
    B2 = _MLP_BLK // 2
    wi = w_ref[:].reshape(B2, D)
    xl = jnp.maximum(lax.bitcast_convert_type(wi << 16, jnp.float32), 0.0)
    xh = jnp.maximum(
        lax.bitcast_convert_type(wi & jnp.int32(-65536), jnp.float32), 0.0)
    w1lo = w1_ref[0:D2, :]
    w1hi = w1_ref[D2:, :]
    z = jnp.zeros((D2, MID), jnp.float32)
    wa = jnp.concatenate(
        [jnp.concatenate([w1lo, z], axis=1),
         jnp.concatenate([z, w1lo], axis=1)], axis=0)
    wb = jnp.concatenate(
        [jnp.concatenate([w1hi, z], axis=1),
         jnp.concatenate([z, w1hi], axis=1)], axis=0)
    b1d = jnp.concatenate([b1_ref[:], b1_ref[:]], axis=1)
    y = (jnp.dot(xl, wa, preferred_element_type=jnp.float32)
         + jnp.dot(xh, wb, preferred_element_type=jnp.float32) + b1d)
    x = jnp.maximum(y.reshape(_MLP_BLK, MID), 0.0)
    x = jnp.maximum(
        jnp.dot(x, w2_ref[:], preferred_element_type=jnp.float32) + b2_ref[:],
        0.0)
    out_ref[:] = (
        jnp.dot(x, wo_ref[:], preferred_element_type=jnp.float32) + bo_ref[:])


def _mlp(w, W1, b1, W2, b2, W_out, b_out, n_valid):
    kout = W_out.shape[1]
    return pl.pallas_call(
        _mlp_body,
        grid=(P_SEG // _MLP_BLK,),
        in_specs=[
            pl.BlockSpec((_MLP_BLK // 16, 8, D), lambda i: (i, 0, 0)),
            pl.BlockSpec((MID, MID), lambda i: (0, 0)),
            pl.BlockSpec((1, MID), lambda i: (0, 0)),
            pl.BlockSpec((MID, MID), lambda i: (0, 0)),
            pl.BlockSpec((1, MID), lambda i: (0, 0)),
            pl.BlockSpec((MID, kout), lambda i: (0, 0)),
            pl.BlockSpec((1, kout), lambda i: (0, 0)),
        ],
        out_specs=pl.BlockSpec((_MLP_BLK, kout), lambda i: (i, 0)),
        out_shape=jax.ShapeDtypeStruct((n_valid, kout), jnp.float32),
    )(w, W1, b1, W2, b2, W_out, b_out)


def kernel(h, idx, W0, b0, W1, b1, W2, b2, W_out, b_out):
    u, v = _uv_tables(h, W0, b0.reshape(1, D))

    idxp = jnp.concatenate(
        [idx, jnp.zeros((P - N_IMP, 4), jnp.int32)], axis=0)
    i0, i1, i2, i3 = idxp[:, 0], idxp[:, 1], idxp[:, 2], idxp[:, 3]

    b1r = b1.reshape(1, MID)
    b2r = b2.reshape(1, MID)
    bor = b_out.reshape(1, K_OUT)

    outs = []
    for s in range(NSEG):
        lo = s * P_SEG
        sl = slice(lo, lo + P_SEG)
        w = _sc_gather_combine(u, v, i0[sl], i1[sl], i2[sl], i3[sl])
        w3 = w.reshape(P_SEG // 16, 8, D)
        n_valid = min(P_SEG, N_IMP - lo)
        outs.append(_mlp(w3, W1, b1r, W2, b2r, W_out, bor, n_valid))
    return jnp.concatenate(outs, axis=0)
